# baseline (speedup 1.0000x reference)
import functools

import jax
import jax.numpy as jnp
from jax import lax
from jax.experimental import pallas as pl
from jax.experimental.pallas import tpu as pltpu
from jax.experimental.pallas import tpu_sc as plsc

_NC = 2
_NS = 16
_NW = _NC * _NS
_CH = 64
_GRP = 8

_NEG = -1e30
_SPLIT0 = 0.83


def _edge_e_body(ea_ref, We1_ref, be1_ref, Wlo_ref, blo_ref,
                 Whi_ref, bhi_ref, *outs):
    a = jnp.maximum(
        jnp.dot(ea_ref[...], We1_ref[...], preferred_element_type=jnp.float32)
        + be1_ref[...], 0.0)
    for i in range(len(outs)):
        elo = jnp.dot(a, Wlo_ref[i], preferred_element_type=jnp.float32) \
            + blo_ref[i]
        ehi = jnp.dot(a, Whi_ref[i], preferred_element_type=jnp.float32) \
            + bhi_ref[i]
        lo16 = jax.lax.shift_right_logical(
            jax.lax.bitcast_convert_type(elo, jnp.int32) + 0x8000, 16)
        hi16 = (jax.lax.bitcast_convert_type(ehi, jnp.int32)
                + 0x8000) & jnp.int32(-65536)
        outs[i][...] = lo16 | hi16


def _edge_e(ea, We1, be1, Wlo, blo, Whi, bhi, e_total, e_pad, h):
    r = 2048
    grid = -(-e_total // r)
    hw = h // 2
    nout = Wlo.shape[0]
    out = jax.ShapeDtypeStruct((e_pad, hw), jnp.int32)
    return pl.pallas_call(
        _edge_e_body,
        grid=(grid,),
        in_specs=[
            pl.BlockSpec((r, ea.shape[1]), lambda j: (j, 0)),
            pl.BlockSpec(We1.shape, lambda j: (0, 0)),
            pl.BlockSpec(be1.shape, lambda j: (0, 0)),
            pl.BlockSpec(Wlo.shape, lambda j: (0, 0, 0)),
            pl.BlockSpec(blo.shape, lambda j: (0, 0, 0)),
            pl.BlockSpec(Whi.shape, lambda j: (0, 0, 0)),
            pl.BlockSpec(bhi.shape, lambda j: (0, 0, 0)),
        ],
        out_specs=[pl.BlockSpec((r, hw), lambda j: (j, 0))] * nout,
        out_shape=[out] * nout,
    )(ea, We1, be1, Wlo, blo, Whi, bhi)


def _sc_layer_body(nacc, h, epw0, epw1, h_hbm, e_hbm, src_hbm, dst_hbm,
                   zero_hbm, out_hbm, src_v, dst_v, ebuf, hbuf, acc, *sems):
    cid = lax.axis_index("c")
    sid = lax.axis_index("s")
    stripe = (nacc // (8 * _NS)) * 8
    tail = nacc - _NS * stripe
    row0 = sid * stripe
    pltpu.sync_copy(zero_hbm.at[pl.ds(row0, stripe)], acc.at[pl.ds(row0, stripe)])
    if tail > 0:
        @pl.when(sid == _NS - 1)
        def _zero_tail():
            pltpu.sync_copy(zero_hbm.at[pl.ds(_NS * stripe, tail)],
                            acc.at[pl.ds(_NS * stripe, tail)])
    plsc.subcore_barrier()

    crows0, crows1 = epw0 // _CH, epw1 // _CH
    crow_base = jnp.where(cid == 0, sid * crows0,
                          _NS * crows0 + sid * crows1)
    ngrp = jnp.where(cid == 0, crows0 // _GRP, crows1 // _GRP)
    esems, hsems, ssems = sems[0:2], sems[2:5], sems[5:8]

    def group(gg, _):
        crow = crow_base + gg * _GRP
        pltpu.sync_copy(src_hbm.at[pl.ds(crow, _GRP)], src_v)
        pltpu.sync_copy(dst_hbm.at[pl.ds(crow, _GRP)], dst_v)
        base_g = crow * _CH

        def fetch(k):
            eslot, hslot = k & 1, k % 3
            ed = pltpu.async_copy(e_hbm.at[pl.ds(base_g + k * _CH, _CH)],
                                  ebuf.at[eslot], esems[eslot])
            hd = pltpu.async_copy(h_hbm.at[src_v.at[k]], hbuf.at[hslot],
                                  hsems[hslot])
            return ed, hd

        infl = {k: fetch(k) for k in range(2)}
        scat = {}
        for k in range(_GRP):
            eslot, hslot = k & 1, k % 3
            ed, hd = infl.pop(k)
            ed.wait()
            hd.wait()

            def row(rr, _, eslot=eslot, hslot=hslot):
                shamt = jnp.full((16,), 16, jnp.int32)
                mask = jnp.full((16,), -65536, jnp.int32)
                for m in range(h // 32):
                    w = ebuf[eslot, rr, pl.ds(m * 16, 16)]
                    lo = jax.lax.bitcast_convert_type(
                        jax.lax.shift_left(w, shamt), jnp.float32)
                    hi = jax.lax.bitcast_convert_type(
                        jax.lax.bitwise_and(w, mask), jnp.float32)
                    s0 = pl.ds(m * 32, 16)
                    s1 = pl.ds(m * 32 + 16, 16)
                    hbuf[hslot, rr, s0] = jnp.maximum(
                        hbuf[hslot, rr, s0] + lo, 0.0)
                    hbuf[hslot, rr, s1] = jnp.maximum(
                        hbuf[hslot, rr, s1] + hi, 0.0)
                return ()

            lax.fori_loop(0, _CH, row, ())
            scat[k] = pltpu.async_copy(hbuf.at[hslot], acc.at[dst_v.at[k]],
                                       ssems[hslot], add=True)
            if k + 2 < _GRP:
                if k - 1 in scat:
                    scat.pop(k - 1).wait()
                infl[k + 2] = fetch(k + 2)
        for d in scat.values():
            d.wait()
        return ()

    lax.fori_loop(0, ngrp, group, ())
    plsc.subcore_barrier()
    pltpu.sync_copy(acc.at[pl.ds(row0, stripe)],
                    out_hbm.at[cid, pl.ds(row0, stripe)])
    if tail > 0:
        @pl.when(sid == _NS - 1)
        def _out_tail():
            pltpu.sync_copy(acc.at[pl.ds(_NS * stripe, tail)],
                            out_hbm.at[cid, pl.ds(_NS * stripe, tail)])


def _sc_layer(h_nodes, e_i, src_w, dst_w, zeros_nh, n, nacc, h, epw0, epw1):
    mesh = plsc.VectorSubcoreMesh(core_axis_name="c", subcore_axis_name="s",
                                  num_cores=_NC, num_subcores=_NS)
    kern = pl.kernel(
        functools.partial(_sc_layer_body, nacc, h, epw0, epw1),
        out_type=jax.ShapeDtypeStruct((_NC, nacc, h), jnp.float32),
        mesh=mesh,
        scratch_types=[
            pltpu.VMEM((_GRP, _CH), jnp.int32),
            pltpu.VMEM((_GRP, _CH), jnp.int32),
            pltpu.VMEM((2, _CH, h // 2), jnp.int32),
            pltpu.VMEM((3, _CH, h), jnp.float32),
            pltpu.VMEM_SHARED((nacc, h), jnp.float32),
        ] + [pltpu.SemaphoreType.DMA] * 8,
    )
    return kern(h_nodes, e_i, src_w, dst_w, zeros_nh)


def _node_body(h_ref, a0_ref, a1_ref, W1_ref, b1_ref, W2_ref, b2_ref,
               g_ref, bb_ref, out_ref):
    z = h_ref[...] + a0_ref[...] + a1_ref[...]
    t = jnp.maximum(
        jnp.dot(z, W1_ref[...], preferred_element_type=jnp.float32)
        + b1_ref[...], 0.0)
    t = jnp.dot(t, W2_ref[...], preferred_element_type=jnp.float32) + b2_ref[...]
    mu = jnp.mean(t, axis=-1, keepdims=True)
    var = jnp.mean((t - mu) * (t - mu), axis=-1, keepdims=True)
    t = (t - mu) * jax.lax.rsqrt(var + 1e-5) * g_ref[...] + bb_ref[...]
    out_ref[...] = jnp.maximum(t, 0.0)


def _node_update(h_nodes, a0, a1, W1, b1, W2, b2, g, bb, n, h):
    r = 1000
    return pl.pallas_call(
        _node_body,
        grid=(n // r,),
        in_specs=[
            pl.BlockSpec((r, h), lambda j: (j, 0)),
            pl.BlockSpec((r, h), lambda j: (j, 0)),
            pl.BlockSpec((r, h), lambda j: (j, 0)),
            pl.BlockSpec(W1.shape, lambda j: (0, 0)),
            pl.BlockSpec(b1.shape, lambda j: (0, 0)),
            pl.BlockSpec(W2.shape, lambda j: (0, 0)),
            pl.BlockSpec(b2.shape, lambda j: (0, 0)),
            pl.BlockSpec(g.shape, lambda j: (0, 0)),
            pl.BlockSpec(bb.shape, lambda j: (0, 0)),
        ],
        out_specs=pl.BlockSpec((r, h), lambda j: (j, 0)),
        out_shape=jax.ShapeDtypeStruct((n, h), jnp.float32),
    )(h_nodes, a0, a1, W1, b1, W2, b2, g, bb)


def _pool_body(nblk, g_seg, h_ref, bf_ref, Wg1_ref, bg1_ref, Wg2_ref, bg2_ref,
               Wp1_ref, bp1_ref, plg_ref, plb_ref, Wp2_ref, bp2_ref,
               out_ref, m_s, s_s, num_s):
    p = pl.program_id(0)
    j = pl.program_id(1)
    hb = h_ref[...]
    r = hb.shape[0]
    gate = jnp.maximum(
        jnp.dot(hb, Wg1_ref[...], preferred_element_type=jnp.float32)
        + bg1_ref[...], 0.0)
    gate = jnp.dot(gate, Wg2_ref[...], preferred_element_type=jnp.float32) \
        + bg2_ref[...]
    gid = jax.lax.broadcasted_iota(jnp.int32, (r, g_seg), 1)
    onehot = bf_ref[...] == gid

    @pl.when(p == 0)
    def _phase_max():
        m_blk = jnp.max(jnp.where(onehot, gate, _NEG), axis=0, keepdims=True)

        @pl.when(j == 0)
        def _():
            m_s[...] = m_blk

        @pl.when(j > 0)
        def _():
            m_s[...] = jnp.maximum(m_s[...], m_blk)

    @pl.when(p == 1)
    def _phase_sum():
        msel = jnp.sum(jnp.where(onehot, m_s[...], 0.0), axis=1, keepdims=True)
        ge = jnp.exp(gate - msel)
        A = jnp.where(onehot, ge, 0.0)
        ones = jnp.ones((r, 1), jnp.float32)
        s_blk = jax.lax.dot_general(A, ones, (((0,), (0,)), ((), ())),
                                    preferred_element_type=jnp.float32)
        n_blk = jax.lax.dot_general(A, hb, (((0,), (0,)), ((), ())),
                                    preferred_element_type=jnp.float32)

        @pl.when(j == 0)
        def _():
            s_s[...] = s_blk
            num_s[...] = n_blk

        @pl.when(j > 0)
        def _():
            s_s[...] = s_s[...] + s_blk
            num_s[...] = num_s[...] + n_blk

        @pl.when(j == nblk - 1)
        def _finish():
            gv = num_s[...] / (s_s[...] + 1e-16)
            p1 = jnp.maximum(
                jnp.dot(gv, Wp1_ref[...], preferred_element_type=jnp.float32)
                + bp1_ref[...], 0.0)
            mu = jnp.mean(p1, axis=-1, keepdims=True)
            var = jnp.mean((p1 - mu) * (p1 - mu), axis=-1, keepdims=True)
            pn = (p1 - mu) * jax.lax.rsqrt(var + 1e-5) * plg_ref[...] \
                + plb_ref[...]
            z = jnp.dot(pn, Wp2_ref[...], preferred_element_type=jnp.float32) \
                + bp2_ref[...]
            nrm = jnp.sqrt(jnp.sum(z * z, axis=-1, keepdims=True))
            out_ref[...] = z / jnp.maximum(nrm, 1e-12)


def _pool(h_nodes, batchi, Wg1, bg1, Wg2, bg2, Wp1, bp1, plg, plb, Wp2, bp2,
          n, h, g_seg, out_d):
    r = 1000
    nblk = n // r
    fixed = lambda pj_shape: (lambda p, j: pj_shape)
    return pl.pallas_call(
        functools.partial(_pool_body, nblk, g_seg),
        grid=(2, nblk),
        in_specs=[
            pl.BlockSpec((r, h), lambda p, j: (j, 0)),
            pl.BlockSpec((r, 1), lambda p, j: (j, 0)),
            pl.BlockSpec(Wg1.shape, fixed((0, 0))),
            pl.BlockSpec(bg1.shape, fixed((0, 0))),
            pl.BlockSpec(Wg2.shape, fixed((0, 0))),
            pl.BlockSpec(bg2.shape, fixed((0, 0))),
            pl.BlockSpec(Wp1.shape, fixed((0, 0))),
            pl.BlockSpec(bp1.shape, fixed((0, 0))),
            pl.BlockSpec(plg.shape, fixed((0, 0))),
            pl.BlockSpec(plb.shape, fixed((0, 0))),
            pl.BlockSpec(Wp2.shape, fixed((0, 0))),
            pl.BlockSpec(bp2.shape, fixed((0, 0))),
        ],
        out_specs=pl.BlockSpec((g_seg, out_d), fixed((0, 0))),
        out_shape=jax.ShapeDtypeStruct((g_seg, out_d), jnp.float32),
        scratch_shapes=[
            pltpu.VMEM((1, g_seg), jnp.float32),
            pltpu.VMEM((g_seg, 1), jnp.float32),
            pltpu.VMEM((g_seg, h), jnp.float32),
        ],
    )(h_nodes, batchi, Wg1, bg1, Wg2, bg2, Wp1, bp1, plg, plb, Wp2, bp2)


def kernel(x, edge_index, edge_attr, batch, We1, be1, We2, be2, lin_W, lin_b,
           nnW1, nnb1, nnW2, nnb2, ln_g, ln_b, Wg1, bg1, Wg2, bg2, Wp1, bp1,
           pln_g, pln_b, Wp2, bp2):
    n, h = x.shape
    e_total = edge_index.shape[1]
    g_seg = 256
    out_d = Wp2.shape[1]

    quant = _GRP * _CH
    epw0 = max(quant, int(round(_SPLIT0 * e_total / _NS / quant)) * quant)
    rem = max(0, e_total - _NS * epw0)
    epw1 = max(quant, -(-rem // (_NS * quant)) * quant)
    e_pad = _NS * (epw0 + epw1)
    pad = e_pad - e_total

    W2 = jnp.einsum("hk,ikj->ihj", We2, lin_W)
    b2 = (jnp.einsum("h,ihj->ij", be2, lin_W) + lin_b)[:, None, :]
    u = jnp.arange(h // 2, dtype=jnp.int32)
    lo_cols = 32 * (u // 16) + u % 16
    Wlo, Whi = W2[:, :, lo_cols], W2[:, :, lo_cols + 16]
    blo, bhi = b2[:, :, lo_cols], b2[:, :, lo_cols + 16]

    nacc = n + 8
    src_w = jnp.pad(edge_index[0], (0, pad)).reshape(e_pad // _CH, _CH)
    dst_w = jnp.pad(edge_index[1], (0, pad),
                    constant_values=n).reshape(e_pad // _CH, _CH)
    zeros_nh = jnp.zeros((nacc, h), jnp.float32)
    batchi = batch.astype(jnp.int32)[:, None]

    (e0,) = _edge_e(edge_attr, We1, be1[None, :], Wlo[0:1], blo[0:1],
                    Whi[0:1], bhi[0:1], e_total, e_pad, h)
    e1, e2 = _edge_e(edge_attr, We1, be1[None, :], Wlo[1:3], blo[1:3],
                     Whi[1:3], bhi[1:3], e_total, e_pad, h)
    es = (e0, e1, e2)

    hn = x
    for i in range(3):
        aggr = _sc_layer(hn, es[i], src_w, dst_w, zeros_nh, n, nacc, h,
                         epw0, epw1)
        hn = _node_update(hn, aggr[0], aggr[1], nnW1[i], nnb1[i][None, :],
                          nnW2[i], nnb2[i][None, :], ln_g[i][None, :],
                          ln_b[i][None, :], n, h)

    return _pool(hn, batchi, Wg1, bg1[None, :], Wg2, bg2[None, :],
                 Wp1, bp1[None, :], pln_g[None, :], pln_b[None, :],
                 Wp2, bp2[None, :], n, h, g_seg, out_d)


# speedup vs baseline: 1.0497x; 1.0497x over previous
import functools

import jax
import jax.numpy as jnp
from jax import lax
from jax.experimental import pallas as pl
from jax.experimental.pallas import tpu as pltpu
from jax.experimental.pallas import tpu_sc as plsc

_NC = 2
_NS = 16
_NW = _NC * _NS
_CH = 64
_GRP = 8

_NEG = -1e30
_SPLIT0 = 0.83


def _edge_e_body(ea_ref, We1_ref, be1_ref, Wlo_ref, blo_ref,
                 Whi_ref, bhi_ref, *outs):
    a = jnp.maximum(
        jnp.dot(ea_ref[...], We1_ref[...], preferred_element_type=jnp.float32)
        + be1_ref[...], 0.0)
    for i in range(len(outs)):
        elo = jnp.dot(a, Wlo_ref[i], preferred_element_type=jnp.float32) \
            + blo_ref[i]
        ehi = jnp.dot(a, Whi_ref[i], preferred_element_type=jnp.float32) \
            + bhi_ref[i]
        lo16 = jax.lax.shift_right_logical(
            jax.lax.bitcast_convert_type(elo, jnp.int32) + 0x8000, 16)
        hi16 = (jax.lax.bitcast_convert_type(ehi, jnp.int32)
                + 0x8000) & jnp.int32(-65536)
        outs[i][...] = lo16 | hi16


def _edge_e(ea, We1, be1, Wlo, blo, Whi, bhi, e_total, e_pad, h):
    r = 2048
    grid = -(-e_total // r)
    hw = h // 2
    nout = Wlo.shape[0]
    out = jax.ShapeDtypeStruct((e_pad, hw), jnp.int32)
    return pl.pallas_call(
        _edge_e_body,
        grid=(grid,),
        in_specs=[
            pl.BlockSpec((r, ea.shape[1]), lambda j: (j, 0)),
            pl.BlockSpec(We1.shape, lambda j: (0, 0)),
            pl.BlockSpec(be1.shape, lambda j: (0, 0)),
            pl.BlockSpec(Wlo.shape, lambda j: (0, 0, 0)),
            pl.BlockSpec(blo.shape, lambda j: (0, 0, 0)),
            pl.BlockSpec(Whi.shape, lambda j: (0, 0, 0)),
            pl.BlockSpec(bhi.shape, lambda j: (0, 0, 0)),
        ],
        out_specs=[pl.BlockSpec((r, hw), lambda j: (j, 0))] * nout,
        out_shape=[out] * nout,
    )(ea, We1, be1, Wlo, blo, Whi, bhi)


def _sc_layer_body(nacc, h, epw0, epw1, h_hbm, e_hbm, src_hbm, dst_hbm,
                   zero_hbm, out_hbm, src_v, dst_v, ebuf, hbuf, acc, *sems):
    cid = lax.axis_index("c")
    sid = lax.axis_index("s")
    stripe = (nacc // (8 * _NS)) * 8
    tail = nacc - _NS * stripe
    row0 = sid * stripe
    pltpu.sync_copy(zero_hbm.at[pl.ds(row0, stripe)], acc.at[pl.ds(row0, stripe)])
    if tail > 0:
        @pl.when(sid == _NS - 1)
        def _zero_tail():
            pltpu.sync_copy(zero_hbm.at[pl.ds(_NS * stripe, tail)],
                            acc.at[pl.ds(_NS * stripe, tail)])
    plsc.subcore_barrier()

    crows0, crows1 = epw0 // _CH, epw1 // _CH
    crow_base = jnp.where(cid == 0, sid * crows0,
                          _NS * crows0 + sid * crows1)
    ngrp = jnp.where(cid == 0, crows0 // _GRP, crows1 // _GRP)
    esems, hsems, ssems = sems[0:2], sems[2:5], sems[5:8]

    def group(gg, _):
        crow = crow_base + gg * _GRP
        pltpu.sync_copy(src_hbm.at[pl.ds(crow, _GRP)], src_v)
        pltpu.sync_copy(dst_hbm.at[pl.ds(crow, _GRP)], dst_v)
        base_g = crow * _CH

        def fetch(k):
            eslot, hslot = k & 1, k % 3
            ed = pltpu.async_copy(e_hbm.at[pl.ds(base_g + k * _CH, _CH)],
                                  ebuf.at[eslot], esems[eslot])
            hd = pltpu.async_copy(h_hbm.at[src_v.at[k]], hbuf.at[hslot],
                                  hsems[hslot])
            return ed, hd

        infl = {k: fetch(k) for k in range(2)}
        scat = {}
        for k in range(_GRP):
            eslot, hslot = k & 1, k % 3
            ed, hd = infl.pop(k)
            ed.wait()
            hd.wait()

            def row(rr, _, eslot=eslot, hslot=hslot):
                shamt = jnp.full((16,), 16, jnp.int32)
                mask = jnp.full((16,), -65536, jnp.int32)
                for m in range(h // 32):
                    w = ebuf[eslot, rr, pl.ds(m * 16, 16)]
                    lo = jax.lax.bitcast_convert_type(
                        jax.lax.shift_left(w, shamt), jnp.float32)
                    hi = jax.lax.bitcast_convert_type(
                        jax.lax.bitwise_and(w, mask), jnp.float32)
                    s0 = pl.ds(m * 32, 16)
                    s1 = pl.ds(m * 32 + 16, 16)
                    hbuf[hslot, rr, s0] = jnp.maximum(
                        hbuf[hslot, rr, s0] + lo, 0.0)
                    hbuf[hslot, rr, s1] = jnp.maximum(
                        hbuf[hslot, rr, s1] + hi, 0.0)
                return ()

            lax.fori_loop(0, _CH, row, ())
            scat[k] = pltpu.async_copy(hbuf.at[hslot], acc.at[dst_v.at[k]],
                                       ssems[hslot], add=True)
            if k + 2 < _GRP:
                if k - 1 in scat:
                    scat.pop(k - 1).wait()
                infl[k + 2] = fetch(k + 2)
        for d in scat.values():
            d.wait()
        return ()

    lax.fori_loop(0, ngrp, group, ())
    plsc.subcore_barrier()
    pltpu.sync_copy(acc.at[pl.ds(row0, stripe)],
                    out_hbm.at[cid, pl.ds(row0, stripe)])
    if tail > 0:
        @pl.when(sid == _NS - 1)
        def _out_tail():
            pltpu.sync_copy(acc.at[pl.ds(_NS * stripe, tail)],
                            out_hbm.at[cid, pl.ds(_NS * stripe, tail)])


def _sc_layer(h_nodes, e_i, src_w, dst_w, zeros_nh, n, nacc, h, epw0, epw1):
    mesh = plsc.VectorSubcoreMesh(core_axis_name="c", subcore_axis_name="s",
                                  num_cores=_NC, num_subcores=_NS)
    kern = pl.kernel(
        functools.partial(_sc_layer_body, nacc, h, epw0, epw1),
        out_type=jax.ShapeDtypeStruct((_NC, nacc, h), jnp.float32),
        mesh=mesh,
        scratch_types=[
            pltpu.VMEM((_GRP, _CH), jnp.int32),
            pltpu.VMEM((_GRP, _CH), jnp.int32),
            pltpu.VMEM((2, _CH, h // 2), jnp.int32),
            pltpu.VMEM((3, _CH, h), jnp.float32),
            pltpu.VMEM_SHARED((nacc, h), jnp.float32),
        ] + [pltpu.SemaphoreType.DMA] * 8,
    )
    return kern(h_nodes, e_i, src_w, dst_w, zeros_nh)


def _node_body(h_ref, a0_ref, a1_ref, W1_ref, b1_ref, W2_ref, b2_ref,
               g_ref, bb_ref, out_ref):
    z = h_ref[...] + a0_ref[...] + a1_ref[...]
    t = jnp.maximum(
        jnp.dot(z, W1_ref[...], preferred_element_type=jnp.float32)
        + b1_ref[...], 0.0)
    t = jnp.dot(t, W2_ref[...], preferred_element_type=jnp.float32) + b2_ref[...]
    mu = jnp.mean(t, axis=-1, keepdims=True)
    var = jnp.mean((t - mu) * (t - mu), axis=-1, keepdims=True)
    t = (t - mu) * jax.lax.rsqrt(var + 1e-5) * g_ref[...] + bb_ref[...]
    out_ref[...] = jnp.maximum(t, 0.0)


def _node_update(h_nodes, a0, a1, W1, b1, W2, b2, g, bb, n, h):
    r = 1000
    return pl.pallas_call(
        _node_body,
        grid=(n // r,),
        in_specs=[
            pl.BlockSpec((r, h), lambda j: (j, 0)),
            pl.BlockSpec((r, h), lambda j: (j, 0)),
            pl.BlockSpec((r, h), lambda j: (j, 0)),
            pl.BlockSpec(W1.shape, lambda j: (0, 0)),
            pl.BlockSpec(b1.shape, lambda j: (0, 0)),
            pl.BlockSpec(W2.shape, lambda j: (0, 0)),
            pl.BlockSpec(b2.shape, lambda j: (0, 0)),
            pl.BlockSpec(g.shape, lambda j: (0, 0)),
            pl.BlockSpec(bb.shape, lambda j: (0, 0)),
        ],
        out_specs=pl.BlockSpec((r, h), lambda j: (j, 0)),
        out_shape=jax.ShapeDtypeStruct((n, h), jnp.float32),
    )(h_nodes, a0, a1, W1, b1, W2, b2, g, bb)


def _pool_body(nblk, g_seg, h_ref, bf_ref, Wg1_ref, bg1_ref, Wg2_ref, bg2_ref,
               Wp1_ref, bp1_ref, plg_ref, plb_ref, Wp2_ref, bp2_ref,
               out_ref, m_s, s_s, num_s):
    p = pl.program_id(0)
    j = pl.program_id(1)
    hb = h_ref[...]
    r = hb.shape[0]
    gate = jnp.maximum(
        jnp.dot(hb, Wg1_ref[...], preferred_element_type=jnp.float32)
        + bg1_ref[...], 0.0)
    gate = jnp.dot(gate, Wg2_ref[...], preferred_element_type=jnp.float32) \
        + bg2_ref[...]
    gid = jax.lax.broadcasted_iota(jnp.int32, (r, g_seg), 1)
    onehot = bf_ref[...] == gid

    @pl.when(p == 0)
    def _phase_max():
        m_blk = jnp.max(jnp.where(onehot, gate, _NEG), axis=0, keepdims=True)

        @pl.when(j == 0)
        def _():
            m_s[...] = m_blk

        @pl.when(j > 0)
        def _():
            m_s[...] = jnp.maximum(m_s[...], m_blk)

    @pl.when(p == 1)
    def _phase_sum():
        msel = jnp.sum(jnp.where(onehot, m_s[...], 0.0), axis=1, keepdims=True)
        ge = jnp.exp(gate - msel)
        A = jnp.where(onehot, ge, 0.0)
        ones = jnp.ones((r, 1), jnp.float32)
        s_blk = jax.lax.dot_general(A, ones, (((0,), (0,)), ((), ())),
                                    preferred_element_type=jnp.float32)
        n_blk = jax.lax.dot_general(A, hb, (((0,), (0,)), ((), ())),
                                    preferred_element_type=jnp.float32)

        @pl.when(j == 0)
        def _():
            s_s[...] = s_blk
            num_s[...] = n_blk

        @pl.when(j > 0)
        def _():
            s_s[...] = s_s[...] + s_blk
            num_s[...] = num_s[...] + n_blk

        @pl.when(j == nblk - 1)
        def _finish():
            gv = num_s[...] / (s_s[...] + 1e-16)
            p1 = jnp.maximum(
                jnp.dot(gv, Wp1_ref[...], preferred_element_type=jnp.float32)
                + bp1_ref[...], 0.0)
            mu = jnp.mean(p1, axis=-1, keepdims=True)
            var = jnp.mean((p1 - mu) * (p1 - mu), axis=-1, keepdims=True)
            pn = (p1 - mu) * jax.lax.rsqrt(var + 1e-5) * plg_ref[...] \
                + plb_ref[...]
            z = jnp.dot(pn, Wp2_ref[...], preferred_element_type=jnp.float32) \
                + bp2_ref[...]
            nrm = jnp.sqrt(jnp.sum(z * z, axis=-1, keepdims=True))
            out_ref[...] = z / jnp.maximum(nrm, 1e-12)


def _pool(h_nodes, batchi, Wg1, bg1, Wg2, bg2, Wp1, bp1, plg, plb, Wp2, bp2,
          n, h, g_seg, out_d):
    r = 1000
    nblk = n // r
    fixed = lambda pj_shape: (lambda p, j: pj_shape)
    return pl.pallas_call(
        functools.partial(_pool_body, nblk, g_seg),
        grid=(2, nblk),
        in_specs=[
            pl.BlockSpec((r, h), lambda p, j: (j, 0)),
            pl.BlockSpec((r, 1), lambda p, j: (j, 0)),
            pl.BlockSpec(Wg1.shape, fixed((0, 0))),
            pl.BlockSpec(bg1.shape, fixed((0, 0))),
            pl.BlockSpec(Wg2.shape, fixed((0, 0))),
            pl.BlockSpec(bg2.shape, fixed((0, 0))),
            pl.BlockSpec(Wp1.shape, fixed((0, 0))),
            pl.BlockSpec(bp1.shape, fixed((0, 0))),
            pl.BlockSpec(plg.shape, fixed((0, 0))),
            pl.BlockSpec(plb.shape, fixed((0, 0))),
            pl.BlockSpec(Wp2.shape, fixed((0, 0))),
            pl.BlockSpec(bp2.shape, fixed((0, 0))),
        ],
        out_specs=pl.BlockSpec((g_seg, out_d), fixed((0, 0))),
        out_shape=jax.ShapeDtypeStruct((g_seg, out_d), jnp.float32),
        scratch_shapes=[
            pltpu.VMEM((1, g_seg), jnp.float32),
            pltpu.VMEM((g_seg, 1), jnp.float32),
            pltpu.VMEM((g_seg, h), jnp.float32),
        ],
    )(h_nodes, batchi, Wg1, bg1, Wg2, bg2, Wp1, bp1, plg, plb, Wp2, bp2)


def kernel(x, edge_index, edge_attr, batch, We1, be1, We2, be2, lin_W, lin_b,
           nnW1, nnb1, nnW2, nnb2, ln_g, ln_b, Wg1, bg1, Wg2, bg2, Wp1, bp1,
           pln_g, pln_b, Wp2, bp2):
    n, h = x.shape
    e_total = edge_index.shape[1]
    g_seg = 256
    out_d = Wp2.shape[1]

    quant = _GRP * _CH
    epw0 = max(quant, int(round(_SPLIT0 * e_total / _NS / quant)) * quant)
    rem = max(0, e_total - _NS * epw0)
    epw1 = max(quant, -(-rem // (_NS * quant)) * quant)
    e_pad = _NS * (epw0 + epw1)
    pad = e_pad - e_total

    W2 = jnp.einsum("hk,ikj->ihj", We2, lin_W)
    b2 = (jnp.einsum("h,ihj->ij", be2, lin_W) + lin_b)[:, None, :]
    u = jnp.arange(h // 2, dtype=jnp.int32)
    lo_cols = 32 * (u // 16) + u % 16
    Wlo, Whi = W2[:, :, lo_cols], W2[:, :, lo_cols + 16]
    blo, bhi = b2[:, :, lo_cols], b2[:, :, lo_cols + 16]

    nacc = n + 8
    src_w = jnp.pad(edge_index[0], (0, pad)).reshape(e_pad // _CH, _CH)
    dst_w = jnp.pad(edge_index[1], (0, pad),
                    constant_values=n).reshape(e_pad // _CH, _CH)
    zeros_nh = jnp.zeros((nacc, h), jnp.float32)
    batchi = batch.astype(jnp.int32)[:, None]

    es = _edge_e(edge_attr, We1, be1[None, :], Wlo, blo, Whi, bhi,
                 e_total, e_pad, h)

    hn = x
    for i in range(3):
        aggr = _sc_layer(hn, es[i], src_w, dst_w, zeros_nh, n, nacc, h,
                         epw0, epw1)
        hn = _node_update(hn, aggr[0], aggr[1], nnW1[i], nnb1[i][None, :],
                          nnW2[i], nnb2[i][None, :], ln_g[i][None, :],
                          ln_b[i][None, :], n, h)

    return _pool(hn, batchi, Wg1, bg1[None, :], Wg2, bg2[None, :],
                 Wp1, bp1[None, :], pln_g[None, :], pln_b[None, :],
                 Wp2, bp2[None, :], n, h, g_seg, out_d)


# speedup vs baseline: 1.0717x; 1.0210x over previous
import functools

import jax
import jax.numpy as jnp
from jax import lax
from jax.experimental import pallas as pl
from jax.experimental.pallas import tpu as pltpu
from jax.experimental.pallas import tpu_sc as plsc

_NC = 2
_NS = 16
_NW = _NC * _NS
_CH = 64
_GRP = 8

_NEG = -1e30
_SPLIT0 = 0.86


def _edge_e_body(ea_ref, We1_ref, be1_ref, Wlo_ref, blo_ref,
                 Whi_ref, bhi_ref, *outs):
    a = jnp.maximum(
        jnp.dot(ea_ref[...], We1_ref[...], preferred_element_type=jnp.float32)
        + be1_ref[...], 0.0)
    for i in range(len(outs)):
        elo = jnp.dot(a, Wlo_ref[i], preferred_element_type=jnp.float32) \
            + blo_ref[i]
        ehi = jnp.dot(a, Whi_ref[i], preferred_element_type=jnp.float32) \
            + bhi_ref[i]
        lo16 = jax.lax.shift_right_logical(
            jax.lax.bitcast_convert_type(elo, jnp.int32) + 0x8000, 16)
        hi16 = (jax.lax.bitcast_convert_type(ehi, jnp.int32)
                + 0x8000) & jnp.int32(-65536)
        outs[i][...] = lo16 | hi16


def _edge_e(ea, We1, be1, Wlo, blo, Whi, bhi, e_total, e_pad, h):
    r = 2048
    grid = -(-e_total // r)
    hw = h // 2
    nout = Wlo.shape[0]
    out = jax.ShapeDtypeStruct((e_pad, hw), jnp.int32)
    return pl.pallas_call(
        _edge_e_body,
        grid=(grid,),
        in_specs=[
            pl.BlockSpec((r, ea.shape[1]), lambda j: (j, 0)),
            pl.BlockSpec(We1.shape, lambda j: (0, 0)),
            pl.BlockSpec(be1.shape, lambda j: (0, 0)),
            pl.BlockSpec(Wlo.shape, lambda j: (0, 0, 0)),
            pl.BlockSpec(blo.shape, lambda j: (0, 0, 0)),
            pl.BlockSpec(Whi.shape, lambda j: (0, 0, 0)),
            pl.BlockSpec(bhi.shape, lambda j: (0, 0, 0)),
        ],
        out_specs=[pl.BlockSpec((r, hw), lambda j: (j, 0))] * nout,
        out_shape=[out] * nout,
    )(ea, We1, be1, Wlo, blo, Whi, bhi)


def _sc_layer_body(nacc, h, epw0, epw1, h_hbm, e_hbm, src_hbm, dst_hbm,
                   zero_hbm, out_hbm, src_v, dst_v, ebuf, hbuf, acc, *sems):
    cid = lax.axis_index("c")
    sid = lax.axis_index("s")
    stripe = (nacc // (8 * _NS)) * 8
    tail = nacc - _NS * stripe
    row0 = sid * stripe
    pltpu.sync_copy(zero_hbm.at[pl.ds(row0, stripe)], acc.at[pl.ds(row0, stripe)])
    if tail > 0:
        @pl.when(sid == _NS - 1)
        def _zero_tail():
            pltpu.sync_copy(zero_hbm.at[pl.ds(_NS * stripe, tail)],
                            acc.at[pl.ds(_NS * stripe, tail)])
    plsc.subcore_barrier()

    crows0, crows1 = epw0 // _CH, epw1 // _CH
    crow_base = jnp.where(cid == 0, sid * crows0,
                          _NS * crows0 + sid * crows1)
    ngrp = jnp.where(cid == 0, crows0 // _GRP, crows1 // _GRP)
    esems, hsems, ssems = sems[0:2], sems[2:5], sems[5:8]

    def group(gg, _):
        crow = crow_base + gg * _GRP
        pltpu.sync_copy(src_hbm.at[pl.ds(crow, _GRP)], src_v)
        pltpu.sync_copy(dst_hbm.at[pl.ds(crow, _GRP)], dst_v)
        base_g = crow * _CH

        def fetch(k):
            eslot, hslot = k & 1, k % 3
            ed = pltpu.async_copy(e_hbm.at[pl.ds(base_g + k * _CH, _CH)],
                                  ebuf.at[eslot], esems[eslot])
            hd = pltpu.async_copy(h_hbm.at[src_v.at[k]], hbuf.at[hslot],
                                  hsems[hslot])
            return ed, hd

        infl = {k: fetch(k) for k in range(2)}
        scat = {}
        for k in range(_GRP):
            eslot, hslot = k & 1, k % 3
            ed, hd = infl.pop(k)
            ed.wait()
            hd.wait()

            def row(rr, _, eslot=eslot, hslot=hslot):
                shamt = jnp.full((16,), 16, jnp.int32)
                mask = jnp.full((16,), -65536, jnp.int32)
                for m in range(h // 32):
                    w = ebuf[eslot, rr, pl.ds(m * 16, 16)]
                    lo = jax.lax.bitcast_convert_type(
                        jax.lax.shift_left(w, shamt), jnp.float32)
                    hi = jax.lax.bitcast_convert_type(
                        jax.lax.bitwise_and(w, mask), jnp.float32)
                    s0 = pl.ds(m * 32, 16)
                    s1 = pl.ds(m * 32 + 16, 16)
                    hbuf[hslot, rr, s0] = jnp.maximum(
                        hbuf[hslot, rr, s0] + lo, 0.0)
                    hbuf[hslot, rr, s1] = jnp.maximum(
                        hbuf[hslot, rr, s1] + hi, 0.0)
                return ()

            lax.fori_loop(0, _CH, row, ())
            scat[k] = pltpu.async_copy(hbuf.at[hslot], acc.at[dst_v.at[k]],
                                       ssems[hslot], add=True)
            if k + 2 < _GRP:
                if k - 1 in scat:
                    scat.pop(k - 1).wait()
                infl[k + 2] = fetch(k + 2)
        for d in scat.values():
            d.wait()
        return ()

    lax.fori_loop(0, ngrp, group, ())
    plsc.subcore_barrier()
    pltpu.sync_copy(acc.at[pl.ds(row0, stripe)],
                    out_hbm.at[cid, pl.ds(row0, stripe)])
    if tail > 0:
        @pl.when(sid == _NS - 1)
        def _out_tail():
            pltpu.sync_copy(acc.at[pl.ds(_NS * stripe, tail)],
                            out_hbm.at[cid, pl.ds(_NS * stripe, tail)])


def _sc_layer(h_nodes, e_i, src_w, dst_w, zeros_nh, n, nacc, h, epw0, epw1):
    mesh = plsc.VectorSubcoreMesh(core_axis_name="c", subcore_axis_name="s",
                                  num_cores=_NC, num_subcores=_NS)
    kern = pl.kernel(
        functools.partial(_sc_layer_body, nacc, h, epw0, epw1),
        out_type=jax.ShapeDtypeStruct((_NC, nacc, h), jnp.float32),
        mesh=mesh,
        scratch_types=[
            pltpu.VMEM((_GRP, _CH), jnp.int32),
            pltpu.VMEM((_GRP, _CH), jnp.int32),
            pltpu.VMEM((2, _CH, h // 2), jnp.int32),
            pltpu.VMEM((3, _CH, h), jnp.float32),
            pltpu.VMEM_SHARED((nacc, h), jnp.float32),
        ] + [pltpu.SemaphoreType.DMA] * 8,
    )
    return kern(h_nodes, e_i, src_w, dst_w, zeros_nh)


def _node_body(h_ref, a0_ref, a1_ref, W1_ref, b1_ref, W2_ref, b2_ref,
               g_ref, bb_ref, out_ref):
    z = h_ref[...] + a0_ref[...] + a1_ref[...]
    t = jnp.maximum(
        jnp.dot(z, W1_ref[...], preferred_element_type=jnp.float32)
        + b1_ref[...], 0.0)
    t = jnp.dot(t, W2_ref[...], preferred_element_type=jnp.float32) + b2_ref[...]
    mu = jnp.mean(t, axis=-1, keepdims=True)
    var = jnp.mean((t - mu) * (t - mu), axis=-1, keepdims=True)
    t = (t - mu) * jax.lax.rsqrt(var + 1e-5) * g_ref[...] + bb_ref[...]
    out_ref[...] = jnp.maximum(t, 0.0)


def _node_update(h_nodes, a0, a1, W1, b1, W2, b2, g, bb, n, h):
    r = 1000
    return pl.pallas_call(
        _node_body,
        grid=(n // r,),
        in_specs=[
            pl.BlockSpec((r, h), lambda j: (j, 0)),
            pl.BlockSpec((r, h), lambda j: (j, 0)),
            pl.BlockSpec((r, h), lambda j: (j, 0)),
            pl.BlockSpec(W1.shape, lambda j: (0, 0)),
            pl.BlockSpec(b1.shape, lambda j: (0, 0)),
            pl.BlockSpec(W2.shape, lambda j: (0, 0)),
            pl.BlockSpec(b2.shape, lambda j: (0, 0)),
            pl.BlockSpec(g.shape, lambda j: (0, 0)),
            pl.BlockSpec(bb.shape, lambda j: (0, 0)),
        ],
        out_specs=pl.BlockSpec((r, h), lambda j: (j, 0)),
        out_shape=jax.ShapeDtypeStruct((n, h), jnp.float32),
    )(h_nodes, a0, a1, W1, b1, W2, b2, g, bb)


def _pool_body(nblk, g_seg, h_ref, bf_ref, Wg1_ref, bg1_ref, Wg2_ref, bg2_ref,
               Wp1_ref, bp1_ref, plg_ref, plb_ref, Wp2_ref, bp2_ref,
               out_ref, m_s, s_s, num_s):
    p = pl.program_id(0)
    j = pl.program_id(1)
    hb = h_ref[...]
    r = hb.shape[0]
    gate = jnp.maximum(
        jnp.dot(hb, Wg1_ref[...], preferred_element_type=jnp.float32)
        + bg1_ref[...], 0.0)
    gate = jnp.dot(gate, Wg2_ref[...], preferred_element_type=jnp.float32) \
        + bg2_ref[...]
    gid = jax.lax.broadcasted_iota(jnp.int32, (r, g_seg), 1)
    onehot = bf_ref[...] == gid

    @pl.when(p == 0)
    def _phase_max():
        m_blk = jnp.max(jnp.where(onehot, gate, _NEG), axis=0, keepdims=True)

        @pl.when(j == 0)
        def _():
            m_s[...] = m_blk

        @pl.when(j > 0)
        def _():
            m_s[...] = jnp.maximum(m_s[...], m_blk)

    @pl.when(p == 1)
    def _phase_sum():
        msel = jnp.sum(jnp.where(onehot, m_s[...], 0.0), axis=1, keepdims=True)
        ge = jnp.exp(gate - msel)
        A = jnp.where(onehot, ge, 0.0)
        ones = jnp.ones((r, 1), jnp.float32)
        s_blk = jax.lax.dot_general(A, ones, (((0,), (0,)), ((), ())),
                                    preferred_element_type=jnp.float32)
        n_blk = jax.lax.dot_general(A, hb, (((0,), (0,)), ((), ())),
                                    preferred_element_type=jnp.float32)

        @pl.when(j == 0)
        def _():
            s_s[...] = s_blk
            num_s[...] = n_blk

        @pl.when(j > 0)
        def _():
            s_s[...] = s_s[...] + s_blk
            num_s[...] = num_s[...] + n_blk

        @pl.when(j == nblk - 1)
        def _finish():
            gv = num_s[...] / (s_s[...] + 1e-16)
            p1 = jnp.maximum(
                jnp.dot(gv, Wp1_ref[...], preferred_element_type=jnp.float32)
                + bp1_ref[...], 0.0)
            mu = jnp.mean(p1, axis=-1, keepdims=True)
            var = jnp.mean((p1 - mu) * (p1 - mu), axis=-1, keepdims=True)
            pn = (p1 - mu) * jax.lax.rsqrt(var + 1e-5) * plg_ref[...] \
                + plb_ref[...]
            z = jnp.dot(pn, Wp2_ref[...], preferred_element_type=jnp.float32) \
                + bp2_ref[...]
            nrm = jnp.sqrt(jnp.sum(z * z, axis=-1, keepdims=True))
            out_ref[...] = z / jnp.maximum(nrm, 1e-12)


def _pool(h_nodes, batchi, Wg1, bg1, Wg2, bg2, Wp1, bp1, plg, plb, Wp2, bp2,
          n, h, g_seg, out_d):
    r = 1000
    nblk = n // r
    fixed = lambda pj_shape: (lambda p, j: pj_shape)
    return pl.pallas_call(
        functools.partial(_pool_body, nblk, g_seg),
        grid=(2, nblk),
        in_specs=[
            pl.BlockSpec((r, h), lambda p, j: (j, 0)),
            pl.BlockSpec((r, 1), lambda p, j: (j, 0)),
            pl.BlockSpec(Wg1.shape, fixed((0, 0))),
            pl.BlockSpec(bg1.shape, fixed((0, 0))),
            pl.BlockSpec(Wg2.shape, fixed((0, 0))),
            pl.BlockSpec(bg2.shape, fixed((0, 0))),
            pl.BlockSpec(Wp1.shape, fixed((0, 0))),
            pl.BlockSpec(bp1.shape, fixed((0, 0))),
            pl.BlockSpec(plg.shape, fixed((0, 0))),
            pl.BlockSpec(plb.shape, fixed((0, 0))),
            pl.BlockSpec(Wp2.shape, fixed((0, 0))),
            pl.BlockSpec(bp2.shape, fixed((0, 0))),
        ],
        out_specs=pl.BlockSpec((g_seg, out_d), fixed((0, 0))),
        out_shape=jax.ShapeDtypeStruct((g_seg, out_d), jnp.float32),
        scratch_shapes=[
            pltpu.VMEM((1, g_seg), jnp.float32),
            pltpu.VMEM((g_seg, 1), jnp.float32),
            pltpu.VMEM((g_seg, h), jnp.float32),
        ],
    )(h_nodes, batchi, Wg1, bg1, Wg2, bg2, Wp1, bp1, plg, plb, Wp2, bp2)


def kernel(x, edge_index, edge_attr, batch, We1, be1, We2, be2, lin_W, lin_b,
           nnW1, nnb1, nnW2, nnb2, ln_g, ln_b, Wg1, bg1, Wg2, bg2, Wp1, bp1,
           pln_g, pln_b, Wp2, bp2):
    n, h = x.shape
    e_total = edge_index.shape[1]
    g_seg = 256
    out_d = Wp2.shape[1]

    quant = _GRP * _CH
    epw0 = max(quant, int(round(_SPLIT0 * e_total / _NS / quant)) * quant)
    rem = max(0, e_total - _NS * epw0)
    epw1 = max(quant, -(-rem // (_NS * quant)) * quant)
    e_pad = _NS * (epw0 + epw1)
    pad = e_pad - e_total

    W2 = jnp.einsum("hk,ikj->ihj", We2, lin_W)
    b2 = (jnp.einsum("h,ihj->ij", be2, lin_W) + lin_b)[:, None, :]
    u = jnp.arange(h // 2, dtype=jnp.int32)
    lo_cols = 32 * (u // 16) + u % 16
    Wlo, Whi = W2[:, :, lo_cols], W2[:, :, lo_cols + 16]
    blo, bhi = b2[:, :, lo_cols], b2[:, :, lo_cols + 16]

    nacc = n + 8
    src_w = jnp.pad(edge_index[0], (0, pad)).reshape(e_pad // _CH, _CH)
    dst_w = jnp.pad(edge_index[1], (0, pad),
                    constant_values=n).reshape(e_pad // _CH, _CH)
    zeros_nh = jnp.zeros((nacc, h), jnp.float32)
    batchi = batch.astype(jnp.int32)[:, None]

    es = _edge_e(edge_attr, We1, be1[None, :], Wlo, blo, Whi, bhi,
                 e_total, e_pad, h)

    hn = x
    for i in range(3):
        aggr = _sc_layer(hn, es[i], src_w, dst_w, zeros_nh, n, nacc, h,
                         epw0, epw1)
        hn = _node_update(hn, aggr[0], aggr[1], nnW1[i], nnb1[i][None, :],
                          nnW2[i], nnb2[i][None, :], ln_g[i][None, :],
                          ln_b[i][None, :], n, h)

    return _pool(hn, batchi, Wg1, bg1[None, :], Wg2, bg2[None, :],
                 Wp1, bp1[None, :], pln_g[None, :], pln_b[None, :],
                 Wp2, bp2[None, :], n, h, g_seg, out_d)
